# baseline (speedup 1.0000x reference)
import functools

import jax
import jax.numpy as jnp
from jax.experimental import pallas as pl
from jax.experimental.pallas import tpu as pltpu

N = 50000
H = 128
E = 150000

_BR = 512
_NPAD = ((N + _BR - 1) // _BR) * _BR


def _dense_body(csm_ref, x_ref, sa_ref, sb_ref, cnt_ref, basis_ref, ws_ref,
                lw_ref, lb_ref, out_ref, *, rel_a, rel_b):
    v0 = basis_ref[0]
    v1 = basis_ref[1]
    w_a = csm_ref[rel_a, 0] * v0 + csm_ref[rel_a, 1] * v1
    w_b = csm_ref[rel_b, 0] * v0 + csm_ref[rel_b, 1] * v1
    x = x_ref[...]
    agg = (jnp.dot(sa_ref[...], w_a.T, preferred_element_type=jnp.float32)
           + jnp.dot(sb_ref[...], w_b.T, preferred_element_type=jnp.float32))
    deg = jnp.maximum(cnt_ref[:, :1], 1.0)
    h_self = jnp.dot(x, ws_ref[...].T, preferred_element_type=jnp.float32)
    h = jnp.maximum(h_self + agg / deg, 0.0) + x
    mu = jnp.mean(h, axis=-1, keepdims=True)
    d = h - mu
    var = jnp.mean(d * d, axis=-1, keepdims=True)
    out_ref[...] = d * jax.lax.rsqrt(var + 1e-5) * lw_ref[...] + lb_ref[...]


def _dense_stage(csm, x, s_a, s_b, cnt, basis, w_self, lw, lb, rel_a, rel_b):
    grid = (_NPAD // _BR,)
    row = lambda i: (i, 0)
    fixed = lambda i: (0, 0)
    out = pl.pallas_call(
        functools.partial(_dense_body, rel_a=rel_a, rel_b=rel_b),
        grid=grid,
        in_specs=[
            pl.BlockSpec(memory_space=pltpu.SMEM),
            pl.BlockSpec((_BR, H), row),
            pl.BlockSpec((_BR, H), row),
            pl.BlockSpec((_BR, H), row),
            pl.BlockSpec((_BR, 16), row),
            pl.BlockSpec((2, H, H), lambda i: (0, 0, 0)),
            pl.BlockSpec((H, H), fixed),
            pl.BlockSpec((1, H), fixed),
            pl.BlockSpec((1, H), fixed),
        ],
        out_specs=pl.BlockSpec((_BR, H), row),
        out_shape=jax.ShapeDtypeStruct((_NPAD, H), jnp.float32),
    )(csm, x, s_a, s_b, cnt, basis, w_self, lw, lb)
    return out[:N]


def _pad_rows(a):
    return jnp.pad(a, ((0, _NPAD - N), (0, 0)))


def kernel(x_user, x_item, ei_rates, ei_rated_by, ei_follows, ei_similar,
           basis, coeff, W_self_user, W_self_item,
           ln_w_user, ln_b_user, ln_w_item, ln_b_item):
    csm = jax.nn.softmax(coeff, axis=-1)

    def scat(x_src, ei):
        s = jnp.zeros((N, H), jnp.float32).at[ei[1]].add(x_src[ei[0]])
        c = jnp.zeros((N,), jnp.float32).at[ei[1]].add(1.0)
        return s, jnp.broadcast_to(c[:, None], (N, 16))

    s_rates, c_rates = scat(x_user, ei_rates)
    s_ratedby, c_ratedby = scat(x_item, ei_rated_by)
    s_follows, c_follows = scat(x_user, ei_follows)
    s_similar, c_similar = scat(x_item, ei_similar)

    pad16 = lambda a: jnp.pad(a, ((0, _NPAD - N), (0, 0)))
    out_user = _dense_stage(
        csm, _pad_rows(x_user), _pad_rows(s_ratedby), _pad_rows(s_follows),
        pad16(c_ratedby + c_follows), basis, W_self_user,
        ln_w_user[None, :], ln_b_user[None, :], rel_a=1, rel_b=2)
    out_item = _dense_stage(
        csm, _pad_rows(x_item), _pad_rows(s_rates), _pad_rows(s_similar),
        pad16(c_rates + c_similar), basis, W_self_item,
        ln_w_item[None, :], ln_b_item[None, :], rel_a=0, rel_b=3)
    return (out_user, out_item)
